# baseline (speedup 1.0000x reference)
import functools

import jax
import jax.numpy as jnp
from jax import lax
from jax.experimental import pallas as pl
from jax.experimental.pallas import tpu as pltpu
from jax.experimental.pallas import tpu_sc as plsc

DIM = 64
BATCH = 16384
ROWS2 = 500000
PDIM = 2 * DIM
CHUNK = 128

_info = plsc.get_sparse_core_info()
NC, NS, L = _info.num_cores, _info.num_subcores, _info.num_lanes
NW = NC * NS
B_PER_W = BATCH // NW
NCHUNK = B_PER_W // CHUNK
NGROUP = B_PER_W // L


def _make_sc_kernel():
    mesh = plsc.VectorSubcoreMesh(core_axis_name="c", subcore_axis_name="s")

    @functools.partial(
        pl.kernel,
        mesh=mesh,
        out_type=jax.ShapeDtypeStruct((BATCH,), jnp.float32),
        scratch_types=[
            pltpu.VMEM((NCHUNK, CHUNK), jnp.int32),
            pltpu.VMEM((NCHUNK, CHUNK), jnp.int32),
            pltpu.VMEM((NCHUNK, CHUNK), jnp.int32),
            pltpu.VMEM((NCHUNK, CHUNK), jnp.int32),
            pltpu.VMEM((CHUNK, PDIM), jnp.float32),
            pltpu.VMEM((CHUNK, PDIM), jnp.float32),
            pltpu.VMEM((B_PER_W,), jnp.float32),
            pltpu.SemaphoreType.DMA,
        ],
        compiler_params=pltpu.CompilerParams(
            needs_layout_passes=False, use_tc_tiling_on_sc=True),
    )
    def two_tower(u_hbm, i_hbm, ue2_hbm, ie2_hbm, out_hbm,
                  idx_u, idx_i, pair_u, pair_i, rows_u, rows_i, out_v, sem):
        wid = lax.axis_index("s") * NC + lax.axis_index("c")
        base = wid * B_PER_W

        pltpu.sync_copy(u_hbm.at[wid], idx_u)
        pltpu.sync_copy(i_hbm.at[wid], idx_i)

        def pair_body(t, _):
            j = t // (CHUNK // L)
            s = (t % (CHUNK // L)) * L
            pair_u[j, pl.ds(s, L)] = lax.shift_right_logical(
                idx_u[j, pl.ds(s, L)], 1)
            pair_i[j, pl.ds(s, L)] = lax.shift_right_logical(
                idx_i[j, pl.ds(s, L)], 1)
            return 0

        lax.fori_loop(0, NCHUNK * (CHUNK // L), pair_body, 0, unroll=4)

        lane = lax.iota(jnp.int32, L)

        def chunk_body(j, _):
            cu = pltpu.async_copy(ue2_hbm.at[pair_u.at[j]], rows_u, sem)
            ci = pltpu.async_copy(ie2_hbm.at[pair_i.at[j]], rows_i, sem)
            cu.wait()
            ci.wait()

            def group_body(gg, _):
                slot = gg * L + lane
                pu = idx_u[j, pl.ds(gg * L, L)] & 1
                pi = idx_i[j, pl.ds(gg * L, L)] & 1
                col_u = pu * DIM
                col_i = pi * DIM

                def d_body(d, acc):
                    ug = plsc.load_gather(rows_u, [slot, col_u + d])
                    ig = plsc.load_gather(rows_i, [slot, col_i + d])
                    return acc + ug * ig

                acc = lax.fori_loop(0, DIM, d_body,
                                    jnp.zeros((L,), jnp.float32), unroll=8)
                out_v[pl.ds(j * CHUNK + gg * L, L)] = acc
                return 0

            lax.fori_loop(0, CHUNK // L, group_body, 0)
            return 0

        lax.fori_loop(0, NCHUNK, chunk_body, 0)

        pltpu.sync_copy(out_v, out_hbm.at[pl.ds(base, B_PER_W)])

    return two_tower


_sc_kernel = _make_sc_kernel()


def kernel(u, i, user_emb, item_emb):
    u3 = u.astype(jnp.int32).reshape(NW, NCHUNK, CHUNK)
    i3 = i.astype(jnp.int32).reshape(NW, NCHUNK, CHUNK)
    ue2 = user_emb.reshape(ROWS2, PDIM)
    ie2 = item_emb.reshape(ROWS2, PDIM)
    return _sc_kernel(u3, i3, ue2, ie2)


# speedup vs baseline: 2.5821x; 2.5821x over previous
import functools

import jax
import jax.numpy as jnp
from jax import lax
from jax.experimental import pallas as pl
from jax.experimental.pallas import tpu as pltpu
from jax.experimental.pallas import tpu_sc as plsc

DIM = 64
BATCH = 16384
NU = 1000000
PDIM = 128

_info = plsc.get_sparse_core_info()
NC, NS, L = _info.num_cores, _info.num_subcores, _info.num_lanes
NW = NC * NS
B_PER_W = BATCH // NW

RANGE = 31232
W = 768
NFULL = RANGE // W
TAIL = RANGE - NFULL * W
EXTRA_LO = NW * RANGE
EXTRA = 512
TAIL64_LO = EXTRA_LO + EXTRA
LIST_CAP = 1040
NSLOT = 8
ROWS_OUT = BATCH + L
ICHUNK = 1024


def _make_phase1():
    mesh = plsc.VectorSubcoreMesh(core_axis_name="c", subcore_axis_name="s")

    @functools.partial(
        pl.kernel,
        mesh=mesh,
        out_type=(jax.ShapeDtypeStruct((ROWS_OUT, PDIM), jnp.float32),
                  jax.ShapeDtypeStruct((ROWS_OUT, PDIM), jnp.float32)),
        scratch_types=[
            pltpu.VMEM((ICHUNK,), jnp.int32),
            pltpu.VMEM((LIST_CAP,), jnp.int32),
            pltpu.VMEM((LIST_CAP,), jnp.int32),
            pltpu.VMEM((LIST_CAP,), jnp.int32),
            pltpu.VMEM((LIST_CAP,), jnp.int32),
            pltpu.VMEM((2, DIM, W), jnp.float32),
            pltpu.VMEM((DIM, DIM), jnp.float32),
            pltpu.VMEM((NSLOT, L, PDIM), jnp.float32),
            pltpu.VMEM((NSLOT, L), jnp.int32),
            pltpu.SemaphoreType.DMA,
            pltpu.SemaphoreType.DMA,
        ],
        compiler_params=pltpu.CompilerParams(
            needs_layout_passes=False, use_tc_tiling_on_sc=True),
    )
    def phase1(u_hbm, i_hbm, uet_hbm, iet_hbm, ut_hbm, it_hbm,
               urows_hbm, irows_hbm,
               chunk_v, list_r, list_k, wl_r, wl_k, win, tail_v, stage,
               kstage, sem_w, sem_s):
        wid = lax.axis_index("s") * NC + lax.axis_index("c")
        lo = wid * RANGE
        is_last = wid == NW - 1
        hi = jnp.where(is_last, NU, lo + RANGE)
        lane = lax.iota(jnp.int32, L)

        def run_pass(idx_hbm, tab_hbm, tail_hbm, rows_hbm):
            def chunk_scan(c, cur):
                pltpu.sync_copy(idx_hbm.at[pl.ds(c * ICHUNK, ICHUNK)],
                                chunk_v)

                def bin_body(g, cur):
                    v = chunk_v[pl.ds(g * L, L)]
                    kvec = c * ICHUNK + g * L + lane
                    m = (v >= lo) & (v < hi)
                    plsc.store_compressed(list_r.at[pl.ds(cur, L)], v, mask=m)
                    plsc.store_compressed(list_k.at[pl.ds(cur, L)], kvec, mask=m)
                    return cur + plsc.all_reduce_population_count(m)[0]

                return lax.fori_loop(0, ICHUNK // L, bin_body, cur)

            n = lax.fori_loop(0, BATCH // ICHUNK, chunk_scan, 0)
            ngrp = (n + L - 1) // L

            def process_window(buf, c0, size, issued):
                def scan_body(g, cur2):
                    rv = list_r[pl.ds(g * L, L)]
                    kv = list_k[pl.ds(g * L, L)]
                    m = ((rv >= c0) & (rv < c0 + size)
                         & (g * L + lane < n))
                    plsc.store_compressed(wl_r.at[pl.ds(cur2, L)],
                                          rv - c0, mask=m)
                    plsc.store_compressed(wl_k.at[pl.ds(cur2, L)], kv, mask=m)
                    return cur2 + plsc.all_reduce_population_count(m)[0]

                cur2 = lax.fori_loop(0, ngrp, scan_body, 0)

                def grp_body(g, issued):
                    rem = cur2 - g * L
                    m = lane < rem
                    rloc = wl_r[pl.ds(g * L, L)]
                    kv = wl_k[pl.ds(g * L, L)]
                    kpad = jnp.where(m, kv, BATCH + lane)
                    slot = issued % NSLOT

                    @pl.when(issued >= NSLOT)
                    def _():
                        pltpu.make_async_copy(
                            stage.at[0], rows_hbm.at[pl.ds(0, L)],
                            sem_s).wait()

                    kstage[slot] = kpad

                    def d_body(d, _):
                        dvec = jnp.zeros((L,), jnp.int32) + d
                        vals = plsc.load_gather(buf, [dvec, rloc], mask=m)
                        plsc.store_scatter(stage.at[slot], [lane, dvec],
                                           vals, mask=m)
                        return 0

                    lax.fori_loop(0, DIM, d_body, 0, unroll=8)
                    pltpu.async_copy(stage.at[slot],
                                     rows_hbm.at[kstage.at[slot]], sem_s)
                    return issued + 1

                return lax.fori_loop(0, (cur2 + L - 1) // L, grp_body,
                                     issued)

            c00 = pl.multiple_of(lo, 128)
            pltpu.async_copy(tab_hbm.at[:, pl.ds(c00, W)], win.at[0], sem_w)

            def win_body(v, issued):
                @pl.when(v + 1 < NFULL)
                def _():
                    c0n = pl.multiple_of(lo + (v + 1) * W, 128)
                    pltpu.async_copy(tab_hbm.at[:, pl.ds(c0n, W)],
                                     win.at[(v + 1) % 2], sem_w)
                pltpu.make_async_copy(tab_hbm.at[:, pl.ds(0, W)],
                                      win.at[0], sem_w).wait()
                c0 = lo + v * W
                return process_window(win.at[v % 2], c0, W, issued)

            issued = lax.fori_loop(0, NFULL, win_body, 0)

            c0t = pl.multiple_of(lo + NFULL * W, 128)
            pltpu.sync_copy(tab_hbm.at[:, pl.ds(c0t, TAIL)],
                            win.at[0, :, pl.ds(0, TAIL)])
            issued = process_window(win.at[0], lo + NFULL * W, TAIL, issued)

            def extra_pass(issued):
                pltpu.sync_copy(tab_hbm.at[:, pl.ds(EXTRA_LO, EXTRA)],
                                win.at[1, :, pl.ds(0, EXTRA)])
                issued = process_window(win.at[1], EXTRA_LO, EXTRA, issued)
                pltpu.sync_copy(tail_hbm, tail_v)
                return process_window(tail_v, TAIL64_LO, DIM, issued)

            issued = lax.cond(is_last, extra_pass, lambda s: s, issued)

            def drain_body(t, _):
                pltpu.make_async_copy(stage.at[0], rows_hbm.at[pl.ds(0, L)],
                                      sem_s).wait()
                return 0

            lax.fori_loop(0, jnp.minimum(issued, NSLOT), drain_body, 0)

        run_pass(u_hbm, uet_hbm, ut_hbm, urows_hbm)
        run_pass(i_hbm, iet_hbm, it_hbm, irows_hbm)

    return phase1


def _make_phase2():
    mesh = plsc.VectorSubcoreMesh(core_axis_name="c", subcore_axis_name="s")
    HB = B_PER_W // 2

    @functools.partial(
        pl.kernel,
        mesh=mesh,
        out_type=jax.ShapeDtypeStruct((BATCH,), jnp.float32),
        scratch_types=[
            pltpu.VMEM((HB, PDIM), jnp.float32),
            pltpu.VMEM((HB, PDIM), jnp.float32),
            pltpu.VMEM((B_PER_W,), jnp.float32),
            pltpu.SemaphoreType.DMA,
        ],
        compiler_params=pltpu.CompilerParams(
            needs_layout_passes=False, use_tc_tiling_on_sc=True),
    )
    def phase2(urows_hbm, irows_hbm, out_hbm, ru, ri, out_v, sem):
        wid = lax.axis_index("s") * NC + lax.axis_index("c")
        base = wid * B_PER_W
        lane = lax.iota(jnp.int32, L)

        for h in range(2):
            cu = pltpu.async_copy(
                urows_hbm.at[pl.ds(base + h * HB, HB)], ru, sem)
            ci = pltpu.async_copy(
                irows_hbm.at[pl.ds(base + h * HB, HB)], ri, sem)
            cu.wait()
            ci.wait()

            def group_body(g, _):
                slot = g * L + lane

                def d_body(d, acc):
                    dvec = jnp.zeros((L,), jnp.int32) + d
                    ug = plsc.load_gather(ru, [slot, dvec])
                    ig = plsc.load_gather(ri, [slot, dvec])
                    return acc + ug * ig

                acc = lax.fori_loop(0, DIM, d_body,
                                    jnp.zeros((L,), jnp.float32), unroll=8)
                out_v[pl.ds(h * HB + g * L, L)] = acc
                return 0

            lax.fori_loop(0, HB // L, group_body, 0)

        pltpu.sync_copy(out_v, out_hbm.at[pl.ds(base, B_PER_W)])

    return phase2


_phase1 = _make_phase1()
_phase2 = _make_phase2()


def kernel(u, i, user_emb, item_emb):
    u32 = u.astype(jnp.int32)
    i32 = i.astype(jnp.int32)
    ut = user_emb.T[:, TAIL64_LO:]
    it = item_emb.T[:, TAIL64_LO:]
    urows, irows = _phase1(u32, i32, user_emb.T, item_emb.T, ut, it)
    return _phase2(urows, irows)
